# initial kernel scaffold (unmeasured)
import jax
import jax.numpy as jnp
from jax import lax
from jax.experimental import pallas as pl
from jax.experimental.pallas import tpu as pltpu


def kernel(
    x,
):
    def body(*refs):
        pass

    out_shape = jax.ShapeDtypeStruct(..., jnp.float32)
    return pl.pallas_call(body, out_shape=out_shape)(...)


# baseline (device time: 32257 ns/iter reference)
import jax
import jax.numpy as jnp
from jax import lax
from jax.experimental import pallas as pl
from jax.experimental.pallas import tpu as pltpu


def kernel(x):
    m_per, n = x.shape

    def body(x_ref, out_ref, send_buf, recv_buf, send_sem, recv_sem):
        my_x = lax.axis_index("x")
        my_y = lax.axis_index("y")
        my_z = lax.axis_index("z")
        peer = (1 - my_x, my_y, my_z)

        barrier_sem = pltpu.get_barrier_semaphore()
        pl.semaphore_signal(
            barrier_sem, inc=1, device_id=peer,
            device_id_type=pl.DeviceIdType.MESH,
        )
        pl.semaphore_wait(barrier_sem, 1)

        send_buf[...] = x_ref[...].astype(jnp.bfloat16)
        rdma = pltpu.make_async_remote_copy(
            src_ref=send_buf,
            dst_ref=recv_buf,
            send_sem=send_sem,
            recv_sem=recv_sem,
            device_id=peer,
            device_id_type=pl.DeviceIdType.MESH,
        )
        rdma.start()

        out_ref[pl.ds(my_x * m_per, m_per), :] = x_ref[...]

        rdma.wait()
        out_ref[pl.ds((1 - my_x) * m_per, m_per), :] = (
            recv_buf[...].astype(out_ref.dtype)
        )

    return pl.pallas_call(
        body,
        out_shape=jax.ShapeDtypeStruct((2 * m_per, n), x.dtype),
        in_specs=[pl.BlockSpec(memory_space=pltpu.VMEM)],
        out_specs=pl.BlockSpec(memory_space=pltpu.VMEM),
        scratch_shapes=[
            pltpu.VMEM((m_per, n), jnp.bfloat16),
            pltpu.VMEM((m_per, n), jnp.bfloat16),
            pltpu.SemaphoreType.DMA,
            pltpu.SemaphoreType.DMA,
        ],
        compiler_params=pltpu.CompilerParams(collective_id=0),
    )(x)


# device time: 23440 ns/iter; 1.3762x vs baseline; 1.3762x over previous
import jax
import jax.numpy as jnp
from jax import lax
from jax.experimental import pallas as pl
from jax.experimental.pallas import tpu as pltpu

N_CHUNK = 8


def kernel(x):
    m_per, n = x.shape
    half = m_per // 2
    cm = half // N_CHUNK

    def body(x_ref, out_ref, x_send_sems, x_recv_sems, z_send_sems, z_recv_sems):
        my_x = lax.axis_index("x")
        my_y = lax.axis_index("y")
        my_z = lax.axis_index("z")
        peer_x = (1 - my_x, my_y, my_z)
        peer_z = (my_x, my_y, 1 - my_z)

        barrier_sem = pltpu.get_barrier_semaphore()
        for p in (peer_x, peer_z):
            pl.semaphore_signal(
                barrier_sem, inc=1, device_id=p,
                device_id_type=pl.DeviceIdType.MESH,
            )
        pl.semaphore_wait(barrier_sem, 2)

        own_base = my_x * m_per
        far_base = (1 - my_x) * m_per
        send_base = own_base + my_z * half
        xin_base = far_base + my_z * half
        zin_base = far_base + (1 - my_z) * half

        x_out = []
        for c in range(N_CHUNK):
            sl = pl.ds(send_base + c * cm, cm)
            out_ref[sl, :] = x_ref[pl.ds(my_z * half + c * cm, cm), :].astype(
                out_ref.dtype
            )
            r = pltpu.make_async_remote_copy(
                src_ref=out_ref.at[sl],
                dst_ref=out_ref.at[sl],
                send_sem=x_send_sems.at[c],
                recv_sem=x_recv_sems.at[c],
                device_id=peer_x,
                device_id_type=pl.DeviceIdType.MESH,
            )
            r.start()
            x_out.append(r)

        out_ref[pl.ds(own_base + (1 - my_z) * half, half), :] = x_ref[
            pl.ds((1 - my_z) * half, half), :
        ].astype(out_ref.dtype)

        z_out = []
        for c in range(N_CHUNK):
            sl = pl.ds(xin_base + c * cm, cm)
            rin = pltpu.make_async_remote_copy(
                src_ref=out_ref.at[sl],
                dst_ref=out_ref.at[sl],
                send_sem=x_send_sems.at[c],
                recv_sem=x_recv_sems.at[c],
                device_id=peer_x,
                device_id_type=pl.DeviceIdType.MESH,
            )
            rin.wait_recv()
            rz = pltpu.make_async_remote_copy(
                src_ref=out_ref.at[sl],
                dst_ref=out_ref.at[sl],
                send_sem=z_send_sems.at[c],
                recv_sem=z_recv_sems.at[c],
                device_id=peer_z,
                device_id_type=pl.DeviceIdType.MESH,
            )
            rz.start()
            z_out.append(rz)

        for c in range(N_CHUNK):
            sl = pl.ds(zin_base + c * cm, cm)
            rzin = pltpu.make_async_remote_copy(
                src_ref=out_ref.at[sl],
                dst_ref=out_ref.at[sl],
                send_sem=z_send_sems.at[c],
                recv_sem=z_recv_sems.at[c],
                device_id=peer_z,
                device_id_type=pl.DeviceIdType.MESH,
            )
            rzin.wait_recv()

        for r in x_out:
            r.wait_send()
        for r in z_out:
            r.wait_send()

    return pl.pallas_call(
        body,
        out_shape=jax.ShapeDtypeStruct((2 * m_per, n), jnp.bfloat16),
        in_specs=[pl.BlockSpec(memory_space=pltpu.VMEM)],
        out_specs=pl.BlockSpec(memory_space=pltpu.VMEM),
        scratch_shapes=[
            pltpu.SemaphoreType.DMA((N_CHUNK,)),
            pltpu.SemaphoreType.DMA((N_CHUNK,)),
            pltpu.SemaphoreType.DMA((N_CHUNK,)),
            pltpu.SemaphoreType.DMA((N_CHUNK,)),
        ],
        compiler_params=pltpu.CompilerParams(collective_id=0),
    )(x)


# device time: 23415 ns/iter; 1.3776x vs baseline; 1.0011x over previous
import jax
import jax.numpy as jnp
from jax import lax
from jax.experimental import pallas as pl
from jax.experimental.pallas import tpu as pltpu

C = 16
CIN = 4


def kernel(x):
    m_per, n = x.shape
    half = m_per // 2
    cm = half // C
    cin_m = half // CIN

    def body(
        x_hbm, out_hbm,
        in_v, own_bf, xland,
        lin_sems, lin_other, lown, lxout,
        x_send, x_recv, z_send, z_recv,
    ):
        my_x = lax.axis_index("x")
        my_y = lax.axis_index("y")
        my_z = lax.axis_index("z")
        peer_x = (1 - my_x, my_y, my_z)
        peer_z = (my_x, my_y, 1 - my_z)

        own_base = my_x * m_per
        far_base = (1 - my_x) * m_per
        xin_base = far_base + my_z * half
        zin_base = far_base + (1 - my_z) * half

        lin = []
        for ci in range(CIN):
            rows = pl.ds(my_z * half + ci * cin_m, cin_m)
            cp = pltpu.make_async_copy(x_hbm.at[rows, :], in_v.at[rows, :],
                                       lin_sems.at[ci])
            cp.start()
            lin.append(cp)
        rows_o = pl.ds((1 - my_z) * half, half)
        cp_other = pltpu.make_async_copy(x_hbm.at[rows_o, :],
                                         in_v.at[rows_o, :], lin_other)
        cp_other.start()

        barrier_sem = pltpu.get_barrier_semaphore()
        for p in (peer_x, peer_z):
            pl.semaphore_signal(
                barrier_sem, inc=1, device_id=p,
                device_id_type=pl.DeviceIdType.MESH,
            )
        pl.semaphore_wait(barrier_sem, 2)

        x_out = []
        for c in range(C):
            if c % (C // CIN) == 0:
                lin[c // (C // CIN)].wait()
            rows = pl.ds(my_z * half + c * cm, cm)
            own_bf[rows, :] = in_v[rows, :].astype(jnp.bfloat16)
            r = pltpu.make_async_remote_copy(
                src_ref=own_bf.at[rows, :],
                dst_ref=xland.at[pl.ds(c * cm, cm), :],
                send_sem=x_send.at[c],
                recv_sem=x_recv.at[c],
                device_id=peer_x,
                device_id_type=pl.DeviceIdType.MESH,
            )
            r.start()
            x_out.append(r)

        cp_other.wait()
        own_bf[rows_o, :] = in_v[rows_o, :].astype(jnp.bfloat16)
        cp_own_out = pltpu.make_async_copy(
            own_bf, out_hbm.at[pl.ds(own_base, m_per), :], lown
        )
        cp_own_out.start()

        z_fwd = []
        x_hbm_out = []
        for c in range(C):
            land = xland.at[pl.ds(c * cm, cm), :]
            rin = pltpu.make_async_remote_copy(
                src_ref=land, dst_ref=land,
                send_sem=x_send.at[c], recv_sem=x_recv.at[c],
                device_id=peer_x, device_id_type=pl.DeviceIdType.MESH,
            )
            rin.wait_recv()
            out_rows = pl.ds(xin_base + c * cm, cm)
            rz = pltpu.make_async_remote_copy(
                src_ref=land,
                dst_ref=out_hbm.at[out_rows, :],
                send_sem=z_send.at[c],
                recv_sem=z_recv.at[c],
                device_id=peer_z,
                device_id_type=pl.DeviceIdType.MESH,
            )
            rz.start()
            z_fwd.append(rz)
            cp = pltpu.make_async_copy(land, out_hbm.at[out_rows, :], lxout)
            cp.start()
            x_hbm_out.append(cp)

        for c in range(C):
            rows = pl.ds(zin_base + c * cm, cm)
            rzin = pltpu.make_async_remote_copy(
                src_ref=xland.at[pl.ds(c * cm, cm), :],
                dst_ref=out_hbm.at[rows, :],
                send_sem=z_send.at[c], recv_sem=z_recv.at[c],
                device_id=peer_z, device_id_type=pl.DeviceIdType.MESH,
            )
            rzin.wait_recv()

        for r in x_out:
            r.wait_send()
        for r in z_fwd:
            r.wait_send()
        cp_own_out.wait()
        for cp in x_hbm_out:
            cp.wait()

    return pl.pallas_call(
        body,
        out_shape=jax.ShapeDtypeStruct((2 * m_per, n), jnp.bfloat16),
        in_specs=[pl.BlockSpec(memory_space=pl.ANY)],
        out_specs=pl.BlockSpec(memory_space=pl.ANY),
        scratch_shapes=[
            pltpu.VMEM((m_per, n), x.dtype),
            pltpu.VMEM((m_per, n), jnp.bfloat16),
            pltpu.VMEM((half, n), jnp.bfloat16),
            pltpu.SemaphoreType.DMA((CIN,)),
            pltpu.SemaphoreType.DMA,
            pltpu.SemaphoreType.DMA,
            pltpu.SemaphoreType.DMA,
            pltpu.SemaphoreType.DMA((C,)),
            pltpu.SemaphoreType.DMA((C,)),
            pltpu.SemaphoreType.DMA((C,)),
            pltpu.SemaphoreType.DMA((C,)),
        ],
        compiler_params=pltpu.CompilerParams(collective_id=0),
    )(x)


# device time: 20039 ns/iter; 1.6097x vs baseline; 1.1685x over previous
import jax
import jax.numpy as jnp
from jax import lax
from jax.experimental import pallas as pl
from jax.experimental.pallas import tpu as pltpu

CM = 64
CQ = 8
CH = CQ // 2


def kernel(x):
    m_per, n = x.shape
    qr = m_per // 4

    def body(x_ref, out_ref, xs, xr, ys, yr, zs, zr):
        my_x = lax.axis_index("x")
        my_y = lax.axis_index("y")
        my_z = lax.axis_index("z")
        peer_x = (1 - my_x, my_y, my_z)
        peer_y = (my_x, 1 - my_y, my_z)
        peer_z = (my_x, my_y, 1 - my_z)

        my_idx = 2 * my_y + my_z
        d_y = 2 * my_y + (1 - my_z)
        d_z = 2 * (1 - my_y) + my_z
        diag = 2 * (1 - my_y) + (1 - my_z)

        own_base = my_x * m_per
        far_base = (1 - my_x) * m_per

        def rdma(rows, send_sem, recv_sem, peer):
            sl = out_ref.at[pl.ds(rows, CM), :]
            return pltpu.make_async_remote_copy(
                src_ref=sl, dst_ref=sl,
                send_sem=send_sem, recv_sem=recv_sem,
                device_id=peer, device_id_type=pl.DeviceIdType.MESH,
            )

        barrier_sem = pltpu.get_barrier_semaphore()
        for p in (peer_x, peer_y, peer_z):
            pl.semaphore_signal(
                barrier_sem, inc=1, device_id=p,
                device_id_type=pl.DeviceIdType.MESH,
            )
        pl.semaphore_wait(barrier_sem, 3)

        x_out = []
        for c in range(CQ):
            loc = my_idx * qr + c * CM
            out_ref[pl.ds(own_base + loc, CM), :] = x_ref[
                pl.ds(loc, CM), :
            ].astype(jnp.bfloat16)
            r = rdma(own_base + loc, xs.at[c], xr.at[c], peer_x)
            r.start()
            x_out.append(r)

        for q in range(4):
            @pl.when(q != my_idx)
            def _(q=q):
                out_ref[pl.ds(own_base + q * qr, qr), :] = x_ref[
                    q * qr : (q + 1) * qr, :
                ].astype(jnp.bfloat16)

        y_out, z_out = [], []
        for c in range(CQ):
            rows = far_base + my_idx * qr + c * CM
            rdma(rows, xs.at[c], xr.at[c], peer_x).wait_recv()
            ry = rdma(rows, ys.at[c], yr.at[c], peer_y)
            ry.start()
            y_out.append(ry)
            rz = rdma(rows, zs.at[c], zr.at[c], peer_z)
            rz.start()
            z_out.append(rz)

        for k in range(CH):
            rdma(far_base + d_y * qr + k * CM, zs.at[k], zr.at[k],
                 peer_z).wait_recv()
            ry = rdma(far_base + d_y * qr + k * CM,
                      ys.at[CQ + k], yr.at[CQ + k], peer_y)
            ry.start()
            y_out.append(ry)
        for k in range(CH):
            rdma(far_base + d_z * qr + (CH + k) * CM, ys.at[CH + k],
                 yr.at[CH + k], peer_y).wait_recv()
            rz = rdma(far_base + d_z * qr + qr // 2 + k * CM,
                      zs.at[CQ + k], zr.at[CQ + k], peer_z)
            rz.start()
            z_out.append(rz)

        for k in range(CH):
            rdma(far_base + d_z * qr + k * CM, ys.at[k], yr.at[k],
                 peer_y).wait_recv()
        for k in range(CH):
            rdma(far_base + d_y * qr + (CH + k) * CM, zs.at[CH + k],
                 zr.at[CH + k], peer_z).wait_recv()
        for k in range(CH):
            rdma(far_base + diag * qr + k * CM, ys.at[CQ + k],
                 yr.at[CQ + k], peer_y).wait_recv()
        for k in range(CH):
            rdma(far_base + diag * qr + qr // 2 + k * CM, zs.at[CQ + k],
                 zr.at[CQ + k], peer_z).wait_recv()

        for r in x_out + y_out + z_out:
            r.wait_send()

    return pl.pallas_call(
        body,
        out_shape=jax.ShapeDtypeStruct((2 * m_per, n), jnp.bfloat16),
        in_specs=[pl.BlockSpec(memory_space=pltpu.VMEM)],
        out_specs=pl.BlockSpec(memory_space=pltpu.VMEM),
        scratch_shapes=[
            pltpu.SemaphoreType.DMA((CQ,)),
            pltpu.SemaphoreType.DMA((CQ,)),
            pltpu.SemaphoreType.DMA((CQ + CH,)),
            pltpu.SemaphoreType.DMA((CQ + CH,)),
            pltpu.SemaphoreType.DMA((CQ + CH,)),
            pltpu.SemaphoreType.DMA((CQ + CH,)),
        ],
        compiler_params=pltpu.CompilerParams(collective_id=0),
    )(x)
